# Initial kernel scaffold; baseline (speedup 1.0000x reference)
import jax
import jax.numpy as jnp
from jax.experimental import pallas as pl


def kernel(x_t, edge_index_t, edge_attr_t, batch_t, x_f, edge_index_f, edge_attr_f, batch_f, params):
    raise NotImplementedError("write your pallas kernel here")


# speedup vs baseline: 7.3920x; 7.3920x over previous
import functools

import jax
import jax.numpy as jnp
from jax import lax
from jax.experimental import pallas as pl
from jax.experimental.pallas import tpu as pltpu
from jax.experimental.pallas import tpu_sc as plsc

_N = 10000
_E = 640000
_G = 64
_C = 80
_CHUNKS = _E // _C
_NC = 2
_NS = 16
_PT = _CHUNKS // _NS
_BLK = 50
_NB = _PT // _BLK
_WTILES = 10
_RPS = _N // _WTILES

_mesh = plsc.VectorSubcoreMesh(
    core_axis_name="c", subcore_axis_name="s",
    num_cores=_NC, num_subcores=_NS)


def _make_agg(F, NPASS):
    @functools.partial(
        pl.kernel,
        out_type=jax.ShapeDtypeStruct((NPASS, _NC, _N, F), jnp.float32),
        mesh=_mesh,
        scratch_types=[
            pltpu.VMEM((_BLK, _C), jnp.int32),
            pltpu.VMEM((_BLK, _C), jnp.int32),
            pltpu.VMEM((_BLK, _C), jnp.float32),
            pltpu.VMEM((_C, F), jnp.float32),
            pltpu.VMEM_SHARED((_N, F), jnp.float32),
            pltpu.SemaphoreType.DMA,
        ],
        compiler_params=pltpu.CompilerParams(use_tc_tiling_on_sc=False),
    )
    def agg(tab_hbm, src_hbm, dst_hbm, w_hbm, out_hbm,
            src_v, dst_v, w_v, rows_v, acc, sem):
        cid = lax.axis_index("c")
        sid = lax.axis_index("s")
        base = sid * _RPS
        q_full = _RPS // _C
        rem = _RPS - q_full * _C

        for p in range(NPASS):
            def zr(r, _):
                for k in range(F // 16):
                    rows_v[r, pl.ds(k * 16, 16)] = jnp.zeros((16,), jnp.float32)
                return 0
            lax.fori_loop(0, _C, zr, 0)

            @pl.when(sid < _WTILES)
            def _():
                for q in range(q_full):
                    pltpu.sync_copy(rows_v, acc.at[pl.ds(base + q * _C, _C)])
                pltpu.sync_copy(rows_v.at[pl.ds(0, rem)],
                                acc.at[pl.ds(base + q_full * _C, rem)])
            plsc.subcore_barrier()

            def blk(bi, _):
                b0 = bi * _BLK
                pltpu.sync_copy(src_hbm.at[p, cid, sid, pl.ds(b0, _BLK)], src_v)
                pltpu.sync_copy(dst_hbm.at[cid, sid, pl.ds(b0, _BLK)], dst_v)
                pltpu.sync_copy(w_hbm.at[cid, sid, pl.ds(b0, _BLK)], w_v)

                def chunk(i, _):
                    pltpu.async_copy(tab_hbm.at[src_v.at[i]], rows_v, sem).wait()

                    def rowblk(rb, _):
                        r0 = rb * 16
                        wvec = w_v[i, pl.ds(r0, 16)]
                        for t in range(16):
                            wv = wvec[t]
                            for k in range(F // 16):
                                sl = pl.ds(k * 16, 16)
                                rows_v[r0 + t, sl] = rows_v[r0 + t, sl] * wv
                        return 0
                    lax.fori_loop(0, _C // 16, rowblk, 0)
                    pltpu.sync_copy(rows_v, acc.at[dst_v.at[i]], add=True)
                    return 0
                lax.fori_loop(0, _BLK, chunk, 0)
                return 0
            lax.fori_loop(0, _NB, blk, 0)
            plsc.subcore_barrier()

            @pl.when(sid < _WTILES)
            def _():
                pltpu.sync_copy(acc.at[pl.ds(base, _RPS)],
                                out_hbm.at[p, cid, pl.ds(base, _RPS)])
            plsc.subcore_barrier()

    return agg


_agg_16 = _make_agg(16, 1)
_agg_32 = _make_agg(32, 1)
_agg_64_1 = _make_agg(64, 1)
_agg_64_2 = _make_agg(64, 2)


def _stage0_body(degp_ref, x_ref, w1_ref, dinv_ref, h1_ref, hs1_ref):
    deg = degp_ref[:, 0:1] + 1.0
    dinv = lax.rsqrt(jnp.maximum(deg, 1e-12))
    dinv_ref[...] = dinv
    h1 = jnp.dot(x_ref[...], w1_ref[...], preferred_element_type=jnp.float32)
    h1_ref[...] = h1
    hs1_ref[...] = h1 * dinv


def _stage0(degp, x, w1):
    n, fo = x.shape[0], w1.shape[1]
    return pl.pallas_call(
        _stage0_body,
        out_shape=(
            jax.ShapeDtypeStruct((n, 1), jnp.float32),
            jax.ShapeDtypeStruct((n, fo), jnp.float32),
            jax.ShapeDtypeStruct((n, fo), jnp.float32),
        ),
    )(degp, x, w1)


def _combine(agg, h, dinv, b, g, be):
    out = dinv * agg + (dinv * dinv) * h + b
    m = jnp.mean(out, axis=0, keepdims=True)
    v = jnp.mean((out - m) ** 2, axis=0, keepdims=True)
    return jax.nn.relu((out - m) * lax.rsqrt(v + 1e-5) * g + be)


def _mid_body(agg_ref, h_ref, dinv_ref, b_ref, g_ref, be_ref, wn_ref,
              hn_ref, hsn_ref):
    dinv = dinv_ref[...]
    y = _combine(agg_ref[...], h_ref[...], dinv,
                 b_ref[...], g_ref[...], be_ref[...])
    hn = jnp.dot(y, wn_ref[...], preferred_element_type=jnp.float32)
    hn_ref[...] = hn
    hsn_ref[...] = hn * dinv


def _mid(agg, h, dinv, b, g, be, wn):
    n, fo = h.shape[0], wn.shape[1]
    return pl.pallas_call(
        _mid_body,
        out_shape=(
            jax.ShapeDtypeStruct((n, fo), jnp.float32),
            jax.ShapeDtypeStruct((n, fo), jnp.float32),
        ),
    )(agg, h, dinv, b, g, be, wn)


def _final_body(agg_ref, h_ref, dinv_ref, b_ref, g_ref, be_ref, batch_ref,
                wp1_ref, bp1_ref, wp2_ref, bp2_ref,
                pool_ref, z_ref, x3_ref):
    x3 = _combine(agg_ref[...], h_ref[...], dinv_ref[...],
                  b_ref[...], g_ref[...], be_ref[...])
    x3_ref[...] = x3
    gid = lax.broadcasted_iota(jnp.int32, (x3.shape[0], _G), 1)
    onehot = (batch_ref[...] == gid).astype(jnp.float32)
    s = lax.dot_general(onehot, x3, (((0,), (0,)), ((), ())),
                        preferred_element_type=jnp.float32)
    c = jnp.sum(onehot, axis=0)
    pool = s / jnp.maximum(c, 1.0)[:, None]
    pool_ref[...] = pool
    z1 = jax.nn.relu(
        jnp.dot(pool, wp1_ref[...], preferred_element_type=jnp.float32)
        + bp1_ref[...])
    z_ref[...] = (jnp.dot(z1, wp2_ref[...], preferred_element_type=jnp.float32)
                  + bp2_ref[...])


def _final(agg, h, dinv, b, g, be, batch2, wp1, bp1, wp2, bp2):
    n, f = h.shape
    return pl.pallas_call(
        _final_body,
        out_shape=(
            jax.ShapeDtypeStruct((_G, f), jnp.float32),
            jax.ShapeDtypeStruct((_G, f), jnp.float32),
            jax.ShapeDtypeStruct((n, f), jnp.float32),
        ),
    )(agg, h, dinv, b, g, be, batch2, wp1, bp1, wp2, bp2)


def _r1(a):
    return a.reshape(1, -1)


def kernel(x_t, edge_index_t, edge_attr_t, batch_t,
           x_f, edge_index_f, edge_attr_f, batch_f, params):
    p = params

    def eshape(a):
        return a.reshape(_NS, _PT, _C)

    src_all = jnp.stack([eshape(edge_index_t[0]),
                         eshape(edge_index_f[0]) + _N])
    dst_all = jnp.stack([eshape(edge_index_t[1]), eshape(edge_index_f[1])])
    w_all = jnp.stack([eshape(edge_attr_t), eshape(edge_attr_f)])
    src_1 = src_all[None]
    src_2 = jnp.stack([src_all, src_all + 2 * _N])

    ones_tab = jnp.ones((2 * _N, 16), jnp.float32)
    degp = _agg_16(ones_tab, src_1, dst_all, w_all)[0]

    dinv_t, h1_t, hs1_t = _stage0(degp[0], x_t, p['W1t'])
    dinv_f, h1_f, hs1_f = _stage0(degp[1], x_f, p['W1f'])

    stacked1 = jnp.concatenate([hs1_t, hs1_f])
    tab1 = stacked1.reshape(2 * _N, 2, 64).transpose(1, 0, 2).reshape(
        4 * _N, 64)
    agg1 = _agg_64_2(tab1, src_2, dst_all, w_all)
    agg1_t = jnp.concatenate([agg1[0, 0], agg1[1, 0]], axis=1)
    agg1_f = jnp.concatenate([agg1[0, 1], agg1[1, 1]], axis=1)

    h2_t, hs2_t = _mid(agg1_t, h1_t, dinv_t, _r1(p['b1t']),
                       _r1(p['g_bn1t']), _r1(p['be_bn1t']), p['W2t'])
    h2_f, hs2_f = _mid(agg1_f, h1_f, dinv_f, _r1(p['b1f']),
                       _r1(p['g_bn1f']), _r1(p['be_bn1f']), p['W2f'])

    tab2 = jnp.concatenate([hs2_t, hs2_f])
    agg2 = _agg_64_1(tab2, src_1, dst_all, w_all)[0]
    h3_t, hs3_t = _mid(agg2[0], h2_t, dinv_t, _r1(p['b2t']),
                       _r1(p['g_bn2t']), _r1(p['be_bn2t']), p['W3t'])
    h3_f, hs3_f = _mid(agg2[1], h2_f, dinv_f, _r1(p['b2f']),
                       _r1(p['g_bn2f']), _r1(p['be_bn2f']), p['W3f'])

    tab3 = jnp.concatenate([hs3_t, hs3_f])
    agg3 = _agg_32(tab3, src_1, dst_all, w_all)[0]

    h_time, z_time, xt = _final(
        agg3[0], h3_t, dinv_t, _r1(p['b3t']), _r1(p['g_bn3t']),
        _r1(p['be_bn3t']), batch_t.reshape(_N, 1),
        p['Wp1t'], _r1(p['bp1t']), p['Wp2t'], _r1(p['bp2t']))
    h_freq, z_freq, xf = _final(
        agg3[1], h3_f, dinv_f, _r1(p['b3f']), _r1(p['g_bn3f']),
        _r1(p['be_bn3f']), batch_f.reshape(_N, 1),
        p['Wp1f'], _r1(p['bp1f']), p['Wp2f'], _r1(p['bp2f']))

    return (h_time, z_time, h_freq, z_freq, xt, xf)


# speedup vs baseline: 19.8582x; 2.6864x over previous
import functools

import jax
import jax.numpy as jnp
from jax import lax
from jax.experimental import pallas as pl
from jax.experimental.pallas import tpu as pltpu
from jax.experimental.pallas import tpu_sc as plsc

_N = 10000
_E = 640000
_G = 64
_C = 80
_CHUNKS = _E // _C
_NC = 2
_NS = 16
_PT = _CHUNKS // _NS
_BLK = 50
_NB = _PT // _BLK
_WTILES = 10
_RPS = _N // _WTILES

_mesh = plsc.VectorSubcoreMesh(
    core_axis_name="c", subcore_axis_name="s",
    num_cores=_NC, num_subcores=_NS)


def _make_agg(F, NPASS):
    @functools.partial(
        pl.kernel,
        out_type=jax.ShapeDtypeStruct((NPASS, _NC, _N, F), jnp.float32),
        mesh=_mesh,
        scratch_types=[
            pltpu.VMEM((_BLK, _C), jnp.int32),
            pltpu.VMEM((_BLK, _C), jnp.int32),
            pltpu.VMEM((_BLK, _C), jnp.float32),
            pltpu.VMEM((_C, F), jnp.float32),
            pltpu.VMEM((_C, F), jnp.float32),
            pltpu.VMEM((_C, F), jnp.float32),
            pltpu.VMEM((_C, F), jnp.float32),
            pltpu.VMEM_SHARED((_N, F), jnp.float32),
            pltpu.SemaphoreType.DMA,
            pltpu.SemaphoreType.DMA,
            pltpu.SemaphoreType.DMA,
            pltpu.SemaphoreType.DMA,
        ],
        compiler_params=pltpu.CompilerParams(use_tc_tiling_on_sc=False),
    )
    def agg(tab_hbm, src_hbm, dst_hbm, w_hbm, out_hbm,
            src_v, dst_v, w_v, g0, g1, sc0, sc1, acc,
            gs0, gs1, ss0, ss1):
        cid = lax.axis_index("c")
        sid = lax.axis_index("s")
        base = sid * _RPS
        q_full = _RPS // _C
        rem = _RPS - q_full * _C
        GB, SB = (g0, g1), (sc0, sc1)
        GS, SS = (gs0, gs1), (ss0, ss1)

        def issue_gather(i, par):
            pltpu.async_copy(tab_hbm.at[src_v.at[i]], GB[par], GS[par])

        def wait_gather(par):
            pltpu.make_async_copy(tab_hbm.at[src_v.at[0]],
                                  GB[par], GS[par]).wait()

        def issue_scatter(i, par):
            pltpu.async_copy(SB[par], acc.at[dst_v.at[i]], SS[par], add=True)

        def wait_scatter(par):
            pltpu.make_async_copy(SB[par], acc.at[pl.ds(0, _C)],
                                  SS[par]).wait()

        def scale(i, par):
            gbuf, sbuf = GB[par], SB[par]

            def rowblk(rb, _):
                r0 = rb * 16
                wvec = w_v[i, pl.ds(r0, 16)]
                for t in range(16):
                    wv = wvec[t]
                    for k in range(F // 16):
                        sl = pl.ds(k * 16, 16)
                        sbuf[r0 + t, sl] = gbuf[r0 + t, sl] * wv
                return 0
            lax.fori_loop(0, _C // 16, rowblk, 0)

        for p in range(NPASS):
            def zr(r, _):
                for k in range(F // 16):
                    sc0[r, pl.ds(k * 16, 16)] = jnp.zeros((16,), jnp.float32)
                return 0
            lax.fori_loop(0, _C, zr, 0)

            @pl.when(sid < _WTILES)
            def _():
                for q in range(q_full):
                    pltpu.sync_copy(sc0, acc.at[pl.ds(base + q * _C, _C)])
                pltpu.sync_copy(sc0.at[pl.ds(0, rem)],
                                acc.at[pl.ds(base + q_full * _C, rem)])
            plsc.subcore_barrier()

            def blk(bi, _):
                b0 = bi * _BLK
                pltpu.sync_copy(src_hbm.at[p, cid, sid, pl.ds(b0, _BLK)], src_v)
                pltpu.sync_copy(dst_hbm.at[cid, sid, pl.ds(b0, _BLK)], dst_v)
                pltpu.sync_copy(w_hbm.at[cid, sid, pl.ds(b0, _BLK)], w_v)

                issue_gather(0, 0)
                issue_gather(1, 1)
                for par in (0, 1):
                    wait_gather(par)
                    scale(par, par)
                    issue_scatter(par, par)
                    issue_gather(par + 2, par)

                def pair(kk, _):
                    for par in (0, 1):
                        i = kk * 2 + par
                        wait_gather(par)
                        wait_scatter(par)
                        scale(i, par)
                        issue_scatter(i, par)
                        issue_gather(i + 2, par)
                    return 0
                lax.fori_loop(1, _BLK // 2 - 1, pair, 0)

                for par in (0, 1):
                    i = _BLK - 2 + par
                    wait_gather(par)
                    wait_scatter(par)
                    scale(i, par)
                    issue_scatter(i, par)
                wait_scatter(0)
                wait_scatter(1)
                return 0
            lax.fori_loop(0, _NB, blk, 0)
            plsc.subcore_barrier()

            @pl.when(sid < _WTILES)
            def _():
                pltpu.sync_copy(acc.at[pl.ds(base, _RPS)],
                                out_hbm.at[p, cid, pl.ds(base, _RPS)])
            plsc.subcore_barrier()

    return agg


_agg_16 = _make_agg(16, 1)
_agg_32 = _make_agg(32, 1)
_agg_64_1 = _make_agg(64, 1)
_agg_64_2 = _make_agg(64, 2)


def _stage0_body(degp_ref, x_ref, w1_ref, dinv_ref, h1_ref, hs1_ref):
    deg = degp_ref[:, 0:1] + 1.0
    dinv = lax.rsqrt(jnp.maximum(deg, 1e-12))
    dinv_ref[...] = dinv
    h1 = jnp.dot(x_ref[...], w1_ref[...], preferred_element_type=jnp.float32)
    h1_ref[...] = h1
    hs1_ref[...] = h1 * dinv


def _stage0(degp, x, w1):
    n, fo = x.shape[0], w1.shape[1]
    return pl.pallas_call(
        _stage0_body,
        out_shape=(
            jax.ShapeDtypeStruct((n, 1), jnp.float32),
            jax.ShapeDtypeStruct((n, fo), jnp.float32),
            jax.ShapeDtypeStruct((n, fo), jnp.float32),
        ),
    )(degp, x, w1)


def _combine(agg, h, dinv, b, g, be):
    out = dinv * agg + (dinv * dinv) * h + b
    m = jnp.mean(out, axis=0, keepdims=True)
    v = jnp.mean((out - m) ** 2, axis=0, keepdims=True)
    return jax.nn.relu((out - m) * lax.rsqrt(v + 1e-5) * g + be)


def _mid_body(agg_ref, h_ref, dinv_ref, b_ref, g_ref, be_ref, wn_ref,
              hn_ref, hsn_ref):
    dinv = dinv_ref[...]
    y = _combine(agg_ref[...], h_ref[...], dinv,
                 b_ref[...], g_ref[...], be_ref[...])
    hn = jnp.dot(y, wn_ref[...], preferred_element_type=jnp.float32)
    hn_ref[...] = hn
    hsn_ref[...] = hn * dinv


def _mid(agg, h, dinv, b, g, be, wn):
    n, fo = h.shape[0], wn.shape[1]
    return pl.pallas_call(
        _mid_body,
        out_shape=(
            jax.ShapeDtypeStruct((n, fo), jnp.float32),
            jax.ShapeDtypeStruct((n, fo), jnp.float32),
        ),
    )(agg, h, dinv, b, g, be, wn)


def _final_body(agg_ref, h_ref, dinv_ref, b_ref, g_ref, be_ref, batch_ref,
                wp1_ref, bp1_ref, wp2_ref, bp2_ref,
                pool_ref, z_ref, x3_ref):
    x3 = _combine(agg_ref[...], h_ref[...], dinv_ref[...],
                  b_ref[...], g_ref[...], be_ref[...])
    x3_ref[...] = x3
    gid = lax.broadcasted_iota(jnp.int32, (x3.shape[0], _G), 1)
    onehot = (batch_ref[...] == gid).astype(jnp.float32)
    s = lax.dot_general(onehot, x3, (((0,), (0,)), ((), ())),
                        preferred_element_type=jnp.float32)
    c = jnp.sum(onehot, axis=0)
    pool = s / jnp.maximum(c, 1.0)[:, None]
    pool_ref[...] = pool
    z1 = jax.nn.relu(
        jnp.dot(pool, wp1_ref[...], preferred_element_type=jnp.float32)
        + bp1_ref[...])
    z_ref[...] = (jnp.dot(z1, wp2_ref[...], preferred_element_type=jnp.float32)
                  + bp2_ref[...])


def _final(agg, h, dinv, b, g, be, batch2, wp1, bp1, wp2, bp2):
    n, f = h.shape
    return pl.pallas_call(
        _final_body,
        out_shape=(
            jax.ShapeDtypeStruct((_G, f), jnp.float32),
            jax.ShapeDtypeStruct((_G, f), jnp.float32),
            jax.ShapeDtypeStruct((n, f), jnp.float32),
        ),
    )(agg, h, dinv, b, g, be, batch2, wp1, bp1, wp2, bp2)


def _r1(a):
    return a.reshape(1, -1)


def kernel(x_t, edge_index_t, edge_attr_t, batch_t,
           x_f, edge_index_f, edge_attr_f, batch_f, params):
    p = params

    def eshape(a):
        return a.reshape(_NS, _PT, _C)

    src_all = jnp.stack([eshape(edge_index_t[0]),
                         eshape(edge_index_f[0]) + _N])
    dst_all = jnp.stack([eshape(edge_index_t[1]), eshape(edge_index_f[1])])
    w_all = jnp.stack([eshape(edge_attr_t), eshape(edge_attr_f)])
    src_1 = src_all[None]
    src_2 = jnp.stack([src_all, src_all + 2 * _N])

    ones_tab = jnp.ones((2 * _N, 16), jnp.float32)
    degp = _agg_16(ones_tab, src_1, dst_all, w_all)[0]

    dinv_t, h1_t, hs1_t = _stage0(degp[0], x_t, p['W1t'])
    dinv_f, h1_f, hs1_f = _stage0(degp[1], x_f, p['W1f'])

    stacked1 = jnp.concatenate([hs1_t, hs1_f])
    tab1 = stacked1.reshape(2 * _N, 2, 64).transpose(1, 0, 2).reshape(
        4 * _N, 64)
    agg1 = _agg_64_2(tab1, src_2, dst_all, w_all)
    agg1_t = jnp.concatenate([agg1[0, 0], agg1[1, 0]], axis=1)
    agg1_f = jnp.concatenate([agg1[0, 1], agg1[1, 1]], axis=1)

    h2_t, hs2_t = _mid(agg1_t, h1_t, dinv_t, _r1(p['b1t']),
                       _r1(p['g_bn1t']), _r1(p['be_bn1t']), p['W2t'])
    h2_f, hs2_f = _mid(agg1_f, h1_f, dinv_f, _r1(p['b1f']),
                       _r1(p['g_bn1f']), _r1(p['be_bn1f']), p['W2f'])

    tab2 = jnp.concatenate([hs2_t, hs2_f])
    agg2 = _agg_64_1(tab2, src_1, dst_all, w_all)[0]
    h3_t, hs3_t = _mid(agg2[0], h2_t, dinv_t, _r1(p['b2t']),
                       _r1(p['g_bn2t']), _r1(p['be_bn2t']), p['W3t'])
    h3_f, hs3_f = _mid(agg2[1], h2_f, dinv_f, _r1(p['b2f']),
                       _r1(p['g_bn2f']), _r1(p['be_bn2f']), p['W3f'])

    tab3 = jnp.concatenate([hs3_t, hs3_f])
    agg3 = _agg_32(tab3, src_1, dst_all, w_all)[0]

    h_time, z_time, xt = _final(
        agg3[0], h3_t, dinv_t, _r1(p['b3t']), _r1(p['g_bn3t']),
        _r1(p['be_bn3t']), batch_t.reshape(_N, 1),
        p['Wp1t'], _r1(p['bp1t']), p['Wp2t'], _r1(p['bp2t']))
    h_freq, z_freq, xf = _final(
        agg3[1], h3_f, dinv_f, _r1(p['b3f']), _r1(p['g_bn3f']),
        _r1(p['be_bn3f']), batch_f.reshape(_N, 1),
        p['Wp1f'], _r1(p['bp1f']), p['Wp2f'], _r1(p['bp2f']))

    return (h_time, z_time, h_freq, z_freq, xt, xf)
